# baseline (speedup 1.0000x reference)
import jax
import jax.numpy as jnp
from jax import lax
from jax.experimental import pallas as pl
from jax.experimental.pallas import tpu as pltpu

_EPS2 = 1e-24


def _silu(y):
    return y * jax.nn.sigmoid(y)


_TAP_OFF = [((k - 1) // 2 + 1, (k - 1) % 2) for k in range(4)]


def _conv_norm_silu(y_ref, w_ref, b_ref, g_ref, h2, w2, c_out):
    m = h2 * w2
    acc = jnp.zeros((m, c_out), jnp.float32)
    t = 0
    for kh in range(4):
        ch, ph = _TAP_OFF[kh]
        for kw in range(4):
            cw, pw = _TAP_OFF[kw]
            xs = y_ref[ph * 2 + pw, ch:ch + h2, cw:cw + w2, :]
            xs = xs.reshape(m, xs.shape[-1])
            acc = acc + jnp.dot(xs, w_ref[t],
                                preferred_element_type=jnp.float32)
            t += 1
    z = acc + b_ref[...]
    ss = jnp.sum(z * z, axis=1, keepdims=True)
    inv = lax.rsqrt(jnp.maximum(ss, _EPS2))
    y = z * inv * g_ref[...]
    return _silu(y)


def _make_block_kernel(h2, w2, c_out):
    def body(y_ref, w_ref, b_ref, g_ref, o_ref):
        o_ref[...] = _conv_norm_silu(
            y_ref, w_ref, b_ref, g_ref, h2, w2, c_out).astype(o_ref.dtype)
    return body


def _make_block2_kernel(h2, w2, c_out):
    def body(y_ref, w_ref, b_ref, g_ref, w1_ref, b1_ref, o_ref):
        h = _conv_norm_silu(y_ref, w_ref, b_ref, g_ref, h2, w2, c_out)
        z = jnp.dot(h.astype(w1_ref.dtype), w1_ref[...],
                    preferred_element_type=jnp.float32) + b1_ref[...]
        o_ref[...] = _silu(z).astype(o_ref.dtype)
    return body


def _s2d(a):
    n, h, w, c = a.shape
    a = a.reshape(n, h // 2, 2, w // 2, 2, c)
    a = a.transpose(0, 2, 4, 1, 3, 5).reshape(n, 4, h // 2, w // 2, c)
    return jnp.pad(a, ((0, 0), (0, 0), (1, 1), (1, 1), (0, 0)))


def _block(y, w, b, g, fuse1x1=None):
    n, _, hc, wc, c_in = y.shape
    h2, w2 = hc - 2, wc - 2
    c_out = w.shape[0]
    m = h2 * w2
    w_taps = w.transpose(2, 3, 1, 0).reshape(16, c_in, c_out).astype(jnp.bfloat16)
    bb = b.astype(jnp.float32).reshape(1, c_out)
    gg = ((float(c_out) ** 0.5)
          * (g.astype(jnp.float32) + 1.0)).reshape(1, c_out)

    in_specs = [
        pl.BlockSpec((None, 4, hc, wc, c_in), lambda i: (i, 0, 0, 0, 0)),
        pl.BlockSpec((16, c_in, c_out), lambda i: (0, 0, 0)),
        pl.BlockSpec((1, c_out), lambda i: (0, 0)),
        pl.BlockSpec((1, c_out), lambda i: (0, 0)),
    ]
    args = [y, w_taps, bb, gg]
    flops = 2 * n * m * 16 * c_in * c_out
    if fuse1x1 is None:
        body = _make_block_kernel(h2, w2, c_out)
    else:
        w1, b1 = fuse1x1
        w1m = w1.reshape(c_out, c_out).T.astype(jnp.bfloat16)
        in_specs += [
            pl.BlockSpec((c_out, c_out), lambda i: (0, 0)),
            pl.BlockSpec((1, c_out), lambda i: (0, 0)),
        ]
        args += [w1m, b1.astype(jnp.float32).reshape(1, c_out)]
        flops += 2 * n * m * c_out * c_out
        body = _make_block2_kernel(h2, w2, c_out)

    return pl.pallas_call(
        body,
        out_shape=jax.ShapeDtypeStruct((n, m, c_out), jnp.bfloat16),
        grid=(n,),
        in_specs=in_specs,
        out_specs=pl.BlockSpec((None, m, c_out), lambda i: (i, 0, 0)),
        compiler_params=pltpu.CompilerParams(
            dimension_semantics=("parallel",),
        ),
        cost_estimate=pl.CostEstimate(
            flops=flops,
            transcendentals=2 * n * m * c_out,
            bytes_accessed=(y.size + n * m * c_out + 16 * c_in * c_out) * 2,
        ),
    )(*args)


def kernel(layer0_w, layer0_b, block0_w, block0_b, block0_g,
           block1_w, block1_b, block1_g, block2_w, block2_b, block2_g,
           logits_w1, logits_b1, logits_w2, logits_b2, x):
    n = x.shape[0]
    y0 = lax.conv_general_dilated(
        x.astype(jnp.bfloat16), layer0_w.astype(jnp.bfloat16),
        window_strides=(1, 1), padding=((2, 2), (2, 2)),
        dimension_numbers=("NCHW", "OIHW", "NHWC"),
        preferred_element_type=jnp.float32)
    y0 = _silu(y0 + layer0_b).astype(jnp.bfloat16)

    hh, ww = y0.shape[1], y0.shape[2]
    h = _block(_s2d(y0), block0_w, block0_b, block0_g)
    hh, ww = hh // 2, ww // 2
    h = _block(_s2d(h.reshape(n, hh, ww, h.shape[-1])),
               block1_w, block1_b, block1_g)
    hh, ww = hh // 2, ww // 2
    h = _block(_s2d(h.reshape(n, hh, ww, h.shape[-1])),
               block2_w, block2_b, block2_g, fuse1x1=(logits_w1, logits_b1))
    h = h.reshape(n, hh // 2, ww // 2, h.shape[-1])

    preds = lax.conv_general_dilated(
        h, logits_w2.astype(jnp.bfloat16),
        window_strides=(1, 1), padding="VALID",
        dimension_numbers=("NHWC", "OIHW", "NCHW"),
        preferred_element_type=jnp.float32)
    return preds + logits_b2.reshape(1, -1, 1, 1)
